# baseline (speedup 1.0000x reference)
import functools

import jax
import jax.numpy as jnp
from jax import lax
from jax.experimental import pallas as pl
from jax.experimental.pallas import tpu as pltpu
from jax.experimental.pallas import tpu_sc as plsc

_N = 10000
_E = 320000
_D = 64
_NC = 2
_NS = 16
_CW = 128
_CH1 = 160
_CH2 = 80
_EP = _NS * _CH1 * _CW
_NP = 10240
_RPT = _NP // _NS
_ZB = 128

_MESH = plsc.VectorSubcoreMesh(core_axis_name="c", subcore_axis_name="s")


def _zero_acc(zbuf, acc, s):
    @pl.loop(0, _ZB)
    def _(i):
        @pl.loop(0, _D // 16)
        def _(kk):
            zbuf[i, pl.ds(kk * 16, 16)] = jnp.zeros((16,), jnp.float32)

    @pl.loop(0, _RPT // _ZB)
    def _(kk):
        pltpu.sync_copy(zbuf, acc.at[pl.ds(s * _RPT + kk * _ZB, _ZB)])


_NB = 5


def _segsum_loop(table, sidx_v, didx_v, msgs, acc, sems, n_ch, bump):
    def xform(jj):
        @pl.when(bump)
        def _():
            for kk in range(_CW // 16):
                v = sidx_v[jj, pl.ds(kk * 16, 16)]
                sidx_v[jj, pl.ds(kk * 16, 16)] = v + 1

    for b in range(_NB):
        xform(b)
        pltpu.make_async_copy(table.at[sidx_v.at[b]], msgs[b],
                              sems[b]).start()

    @pl.loop(0, n_ch, step=_NB)
    def _(j):
        for b in range(_NB):
            pltpu.make_async_copy(table.at[sidx_v.at[j + b]], msgs[b],
                                  sems[b]).wait()
            pltpu.sync_copy(msgs[b], acc.at[didx_v.at[j + b]], add=True)

            @pl.when(j + _NB + b < n_ch)
            def _():
                xform(j + _NB + b)
                pltpu.make_async_copy(table.at[sidx_v.at[j + _NB + b]],
                                      msgs[b], sems[b]).start()


@functools.partial(
    pl.kernel,
    mesh=_MESH,
    out_type=jax.ShapeDtypeStruct((_NP, _NC * _D), jnp.float32),
    scratch_types=[
        pltpu.VMEM((_CH1, _CW), jnp.int32),
        pltpu.VMEM((_CH1, _CW), jnp.int32),
        pltpu.VMEM((_CW, _D), jnp.float32),
        pltpu.VMEM((_CW, _D), jnp.float32),
        pltpu.VMEM((_CW, _D), jnp.float32),
        pltpu.VMEM((_CW, _D), jnp.float32),
        pltpu.VMEM((_CW, _D), jnp.float32),
        pltpu.VMEM((_ZB, _D), jnp.float32),
        pltpu.VMEM_SHARED((_NP, _D), jnp.float32),
        pltpu.SemaphoreType.DMA,
        pltpu.SemaphoreType.DMA,
        pltpu.SemaphoreType.DMA,
        pltpu.SemaphoreType.DMA,
        pltpu.SemaphoreType.DMA,
        pltpu.SemaphoreType.DMA,
    ],
    compiler_params=pltpu.CompilerParams(use_tc_tiling_on_sc=False),
)
def _sc_layer1(table_h, sidx_h, didx_h, out_h, sidx_v, didx_v, msg0, msg1,
               msg2, msg3, msg4, zbuf, acc, sem0, sem1,
               sem2, sem3, sem4, semi):
    c = lax.axis_index("c")
    s = lax.axis_index("s")

    pltpu.make_async_copy(sidx_h.at[s], sidx_v, semi).start()
    pltpu.make_async_copy(didx_h.at[s], didx_v, semi).start()
    _zero_acc(zbuf, acc, s)
    pltpu.make_async_copy(sidx_h.at[s], sidx_v, semi).wait()
    pltpu.make_async_copy(didx_h.at[s], didx_v, semi).wait()
    plsc.subcore_barrier()

    _segsum_loop(table_h, sidx_v, didx_v,
                 (msg0, msg1, msg2, msg3, msg4), acc,
                 (sem0, sem1, sem2, sem3, sem4), _CH1,
                 c == 1)

    plsc.subcore_barrier()
    pltpu.sync_copy(acc.at[pl.ds(s * _RPT, _RPT)],
                    out_h.at[pl.ds(s * _RPT, _RPT), pl.ds(c * _D, _D)])


@functools.partial(
    pl.kernel,
    mesh=_MESH,
    out_type=jax.ShapeDtypeStruct((_NP, _NC * _D), jnp.float32),
    scratch_types=[
        pltpu.VMEM((_CH2, _CW), jnp.int32),
        pltpu.VMEM((_CH2, _CW), jnp.int32),
        pltpu.VMEM((_CW, _D), jnp.float32),
        pltpu.VMEM((_CW, _D), jnp.float32),
        pltpu.VMEM((_CW, _D), jnp.float32),
        pltpu.VMEM((_CW, _D), jnp.float32),
        pltpu.VMEM((_CW, _D), jnp.float32),
        pltpu.VMEM((_ZB, _D), jnp.float32),
        pltpu.VMEM_SHARED((_NP, _D), jnp.float32),
        pltpu.SemaphoreType.DMA,
        pltpu.SemaphoreType.DMA,
        pltpu.SemaphoreType.DMA,
        pltpu.SemaphoreType.DMA,
        pltpu.SemaphoreType.DMA,
        pltpu.SemaphoreType.DMA,
    ],
    compiler_params=pltpu.CompilerParams(use_tc_tiling_on_sc=False),
)
def _sc_layer2(table_h, sidx_h, didx_h, out_h, sidx_v, didx_v, msg0, msg1,
               msg2, msg3, msg4, zbuf, acc, sem0, sem1,
               sem2, sem3, sem4, semi):
    c = lax.axis_index("c")
    s = lax.axis_index("s")
    wid = c * _NS + s

    pltpu.make_async_copy(
        sidx_h.at[wid // 2, pl.ds((wid % 2) * _CH2, _CH2)], sidx_v,
        semi).start()
    pltpu.make_async_copy(
        didx_h.at[wid // 2, pl.ds((wid % 2) * _CH2, _CH2)], didx_v,
        semi).start()
    _zero_acc(zbuf, acc, s)
    pltpu.make_async_copy(
        sidx_h.at[wid // 2, pl.ds((wid % 2) * _CH2, _CH2)], sidx_v,
        semi).wait()
    pltpu.make_async_copy(
        didx_h.at[wid // 2, pl.ds((wid % 2) * _CH2, _CH2)], didx_v,
        semi).wait()
    plsc.subcore_barrier()

    _segsum_loop(table_h, sidx_v, didx_v,
                 (msg0, msg1, msg2, msg3, msg4), acc,
                 (sem0, sem1, sem2, sem3, sem4), _CH2,
                 False)

    plsc.subcore_barrier()
    pltpu.sync_copy(acc.at[pl.ds(s * _RPT, _RPT)],
                    out_h.at[pl.ds(s * _RPT, _RPT), pl.ds(c * _D, _D)])


def _dense1(agg_ref, norm_ref, w1_ref, b1_ref, w2_ref, o_ref):
    hs = agg_ref[...] * norm_ref[...]
    dn = (((1,), (0,)), ((), ()))
    h = lax.dot_general(hs, w1_ref[...], dn,
                        preferred_element_type=jnp.float32) + b1_ref[...]
    hr = jnp.maximum(h, 0.0)
    p = (lax.dot_general(h, w2_ref[0:128], dn,
                         preferred_element_type=jnp.float32)
         + lax.dot_general(hr, w2_ref[128:256], dn,
                           preferred_element_type=jnp.float32))
    o_ref[:, 0:64] = p
    o_ref[:, 64:128] = p


def _dense2(q_ref, norm_ref, b2_ref, o_ref):
    q = q_ref[:, 0:64] + q_ref[:, 64:128]
    o_ref[...] = q * norm_ref[...] + b2_ref[...]


def kernel(x, edge_index, norm, W1, b1, W2, b2):
    npad = _EP - _E
    src_p = (2 * jnp.concatenate(
        [edge_index[0], (jnp.arange(npad, dtype=jnp.int32) % _N)]
    )).reshape(_NS, _CH1, _CW)
    dst_p = jnp.concatenate(
        [edge_index[1],
         _N + 112 + (jnp.arange(npad, dtype=jnp.int32) % 128)]
    ).reshape(_NS, _CH1, _CW)
    x2 = x.reshape(_NC * _N, _D)
    b1r = b1.reshape(1, -1)
    b2r = b2.reshape(1, -1)

    agg = _sc_layer1(x2, src_p, dst_p)

    B1 = 2000
    p2 = pl.pallas_call(
        _dense1,
        grid=(_N // B1,),
        in_specs=[
            pl.BlockSpec((B1, 128), lambda i: (i, 0)),
            pl.BlockSpec((B1, 1), lambda i: (i, 0)),
            pl.BlockSpec((128, 128), lambda i: (0, 0)),
            pl.BlockSpec((1, 128), lambda i: (0, 0)),
            pl.BlockSpec((256, 64), lambda i: (0, 0)),
        ],
        out_specs=pl.BlockSpec((B1, 128), lambda i: (i, 0)),
        out_shape=jax.ShapeDtypeStruct((_N, 128), jnp.float32),
    )(agg, norm, W1, b1r, W2)

    part2 = _sc_layer2(p2.reshape(_NC * _N, _D), src_p, dst_p)

    B2 = 2000
    out = pl.pallas_call(
        _dense2,
        grid=(_N // B2,),
        in_specs=[
            pl.BlockSpec((B2, 128), lambda i: (i, 0)),
            pl.BlockSpec((B2, 1), lambda i: (i, 0)),
            pl.BlockSpec((1, 64), lambda i: (0, 0)),
        ],
        out_specs=pl.BlockSpec((B2, 64), lambda i: (i, 0)),
        out_shape=jax.ShapeDtypeStruct((_N, 64), jnp.float32),
    )(part2, norm, b2r)

    return out


# speedup vs baseline: 1.0016x; 1.0016x over previous
import functools

import jax
import jax.numpy as jnp
from jax import lax
from jax.experimental import pallas as pl
from jax.experimental.pallas import tpu as pltpu
from jax.experimental.pallas import tpu_sc as plsc

_N = 10000
_E = 320000
_D = 64
_NC = 2
_NS = 16
_CW = 128
_CH1 = 160
_CH2 = 80
_EP = _NS * _CH1 * _CW
_NP = 10240
_RPT = _NP // _NS
_ZB = 128

_MESH = plsc.VectorSubcoreMesh(core_axis_name="c", subcore_axis_name="s")


def _zero_acc(zbuf, acc, s):
    @pl.loop(0, _ZB)
    def _(i):
        @pl.loop(0, _D // 16)
        def _(kk):
            zbuf[i, pl.ds(kk * 16, 16)] = jnp.zeros((16,), jnp.float32)

    @pl.loop(0, _RPT // _ZB)
    def _(kk):
        pltpu.sync_copy(zbuf, acc.at[pl.ds(s * _RPT + kk * _ZB, _ZB)])


_NB = 4


def _segsum_loop(table, sidx_v, didx_v, msgs, acc, sems, n_ch, bump):
    def xform(jj):
        @pl.when(bump)
        def _():
            for kk in range(_CW // 16):
                v = sidx_v[jj, pl.ds(kk * 16, 16)]
                sidx_v[jj, pl.ds(kk * 16, 16)] = v + 1

    for b in range(_NB):
        xform(b)
        pltpu.make_async_copy(table.at[sidx_v.at[b]], msgs[b],
                              sems[b]).start()

    @pl.loop(0, n_ch, step=_NB)
    def _(j):
        for b in range(_NB):
            pltpu.make_async_copy(table.at[sidx_v.at[j + b]], msgs[b],
                                  sems[b]).wait()
            pltpu.sync_copy(msgs[b], acc.at[didx_v.at[j + b]], add=True)

            @pl.when(j + _NB + b < n_ch)
            def _():
                xform(j + _NB + b)
                pltpu.make_async_copy(table.at[sidx_v.at[j + _NB + b]],
                                      msgs[b], sems[b]).start()


@functools.partial(
    pl.kernel,
    mesh=_MESH,
    out_type=jax.ShapeDtypeStruct((_NP, _NC * _D), jnp.float32),
    scratch_types=[
        pltpu.VMEM((_CH1, _CW), jnp.int32),
        pltpu.VMEM((_CH1, _CW), jnp.int32),
        pltpu.VMEM((_CW, _D), jnp.float32),
        pltpu.VMEM((_CW, _D), jnp.float32),
        pltpu.VMEM((_CW, _D), jnp.float32),
        pltpu.VMEM((_CW, _D), jnp.float32),
        pltpu.VMEM((_ZB, _D), jnp.float32),
        pltpu.VMEM_SHARED((_NP, _D), jnp.float32),
        pltpu.SemaphoreType.DMA,
        pltpu.SemaphoreType.DMA,
        pltpu.SemaphoreType.DMA,
        pltpu.SemaphoreType.DMA,
        pltpu.SemaphoreType.DMA,
    ],
    compiler_params=pltpu.CompilerParams(use_tc_tiling_on_sc=False),
)
def _sc_layer1(table_h, sidx_h, didx_h, out_h, sidx_v, didx_v, msg0, msg1,
               msg2, msg3, zbuf, acc, sem0, sem1, sem2, sem3, semi):
    c = lax.axis_index("c")
    s = lax.axis_index("s")

    pltpu.make_async_copy(sidx_h.at[s], sidx_v, semi).start()
    pltpu.make_async_copy(didx_h.at[s], didx_v, semi).start()
    _zero_acc(zbuf, acc, s)
    pltpu.make_async_copy(sidx_h.at[s], sidx_v, semi).wait()
    pltpu.make_async_copy(didx_h.at[s], didx_v, semi).wait()
    plsc.subcore_barrier()

    _segsum_loop(table_h, sidx_v, didx_v, (msg0, msg1, msg2, msg3), acc,
                 (sem0, sem1, sem2, sem3), _CH1, c == 1)

    plsc.subcore_barrier()
    pltpu.sync_copy(acc.at[pl.ds(s * _RPT, _RPT)],
                    out_h.at[pl.ds(s * _RPT, _RPT), pl.ds(c * _D, _D)])


@functools.partial(
    pl.kernel,
    mesh=_MESH,
    out_type=jax.ShapeDtypeStruct((_NP, _NC * _D), jnp.float32),
    scratch_types=[
        pltpu.VMEM((_CH2, _CW), jnp.int32),
        pltpu.VMEM((_CH2, _CW), jnp.int32),
        pltpu.VMEM((_CW, _D), jnp.float32),
        pltpu.VMEM((_CW, _D), jnp.float32),
        pltpu.VMEM((_CW, _D), jnp.float32),
        pltpu.VMEM((_CW, _D), jnp.float32),
        pltpu.VMEM((_ZB, _D), jnp.float32),
        pltpu.VMEM_SHARED((_NP, _D), jnp.float32),
        pltpu.SemaphoreType.DMA,
        pltpu.SemaphoreType.DMA,
        pltpu.SemaphoreType.DMA,
        pltpu.SemaphoreType.DMA,
        pltpu.SemaphoreType.DMA,
    ],
    compiler_params=pltpu.CompilerParams(use_tc_tiling_on_sc=False),
)
def _sc_layer2(table_h, sidx_h, didx_h, out_h, sidx_v, didx_v, msg0, msg1,
               msg2, msg3, zbuf, acc, sem0, sem1, sem2, sem3, semi):
    c = lax.axis_index("c")
    s = lax.axis_index("s")
    wid = c * _NS + s

    pltpu.make_async_copy(
        sidx_h.at[wid // 2, pl.ds((wid % 2) * _CH2, _CH2)], sidx_v,
        semi).start()
    pltpu.make_async_copy(
        didx_h.at[wid // 2, pl.ds((wid % 2) * _CH2, _CH2)], didx_v,
        semi).start()
    _zero_acc(zbuf, acc, s)
    pltpu.make_async_copy(
        sidx_h.at[wid // 2, pl.ds((wid % 2) * _CH2, _CH2)], sidx_v,
        semi).wait()
    pltpu.make_async_copy(
        didx_h.at[wid // 2, pl.ds((wid % 2) * _CH2, _CH2)], didx_v,
        semi).wait()
    plsc.subcore_barrier()

    _segsum_loop(table_h, sidx_v, didx_v, (msg0, msg1, msg2, msg3), acc,
                 (sem0, sem1, sem2, sem3), _CH2, False)

    plsc.subcore_barrier()
    pltpu.sync_copy(acc.at[pl.ds(s * _RPT, _RPT)],
                    out_h.at[pl.ds(s * _RPT, _RPT), pl.ds(c * _D, _D)])


def _dense1(agg_ref, norm_ref, w1_ref, b1_ref, w2_ref, o_ref):
    hs = agg_ref[...] * norm_ref[...]
    dn = (((1,), (0,)), ((), ()))
    h = lax.dot_general(hs, w1_ref[...], dn,
                        preferred_element_type=jnp.float32) + b1_ref[...]
    hr = jnp.maximum(h, 0.0)
    p = (lax.dot_general(h, w2_ref[0:128], dn,
                         preferred_element_type=jnp.float32)
         + lax.dot_general(hr, w2_ref[128:256], dn,
                           preferred_element_type=jnp.float32))
    o_ref[:, 0:64] = p
    o_ref[:, 64:128] = p


def _dense2(q_ref, norm_ref, b2_ref, o_ref):
    q = q_ref[:, 0:64] + q_ref[:, 64:128]
    o_ref[...] = q * norm_ref[...] + b2_ref[...]


def kernel(x, edge_index, norm, W1, b1, W2, b2):
    npad = _EP - _E
    src_p = (2 * jnp.concatenate(
        [edge_index[0], (jnp.arange(npad, dtype=jnp.int32) % _N)]
    )).reshape(_NS, _CH1, _CW)
    dst_p = jnp.concatenate(
        [edge_index[1],
         _N + 112 + (jnp.arange(npad, dtype=jnp.int32) % 128)]
    ).reshape(_NS, _CH1, _CW)
    x2 = x.reshape(_NC * _N, _D)
    b1r = b1.reshape(1, -1)
    b2r = b2.reshape(1, -1)

    agg = _sc_layer1(x2, src_p, dst_p)

    B1 = 2000
    p2 = pl.pallas_call(
        _dense1,
        grid=(_N // B1,),
        in_specs=[
            pl.BlockSpec((B1, 128), lambda i: (i, 0)),
            pl.BlockSpec((B1, 1), lambda i: (i, 0)),
            pl.BlockSpec((128, 128), lambda i: (0, 0)),
            pl.BlockSpec((1, 128), lambda i: (0, 0)),
            pl.BlockSpec((256, 64), lambda i: (0, 0)),
        ],
        out_specs=pl.BlockSpec((B1, 128), lambda i: (i, 0)),
        out_shape=jax.ShapeDtypeStruct((_N, 128), jnp.float32),
    )(agg, norm, W1, b1r, W2)

    part2 = _sc_layer2(p2.reshape(_NC * _N, _D), src_p, dst_p)

    B2 = 2000
    out = pl.pallas_call(
        _dense2,
        grid=(_N // B2,),
        in_specs=[
            pl.BlockSpec((B2, 128), lambda i: (i, 0)),
            pl.BlockSpec((B2, 1), lambda i: (i, 0)),
            pl.BlockSpec((1, 64), lambda i: (0, 0)),
        ],
        out_specs=pl.BlockSpec((B2, 64), lambda i: (i, 0)),
        out_shape=jax.ShapeDtypeStruct((_N, 64), jnp.float32),
    )(part2, norm, b2r)

    return out
